# baseline (speedup 1.0000x reference)
import jax, jax.numpy as jnp
from jax.experimental import pallas as pl
from jax.experimental.pallas import tpu as pltpu

_N = 6400000
_K = 8
_SLICE = _N // _K

def _copy_body(ent_in, rel_in, ent_out, rel_out, sems, rsem):
    copies = []
    for k in range(_K):
        c = pltpu.make_async_copy(ent_in.at[pl.ds(k * _SLICE, _SLICE)],
                                  ent_out.at[pl.ds(k * _SLICE, _SLICE)],
                                  sems.at[k])
        c.start(); copies.append(c)
    rc = pltpu.make_async_copy(rel_in, rel_out, rsem)
    rc.start()
    for c in copies:
        c.wait()
    rc.wait()

def kernel(x_dict, edge_index, entity_emb, rel_emb):
    e1 = entity_emb.reshape(_N)
    r1 = rel_emb.reshape(512 * 64)
    eo, ro = pl.pallas_call(
        _copy_body,
        in_specs=[pl.BlockSpec(memory_space=pl.ANY), pl.BlockSpec(memory_space=pl.ANY)],
        out_specs=[pl.BlockSpec(memory_space=pl.ANY), pl.BlockSpec(memory_space=pl.ANY)],
        scratch_shapes=[pltpu.SemaphoreType.DMA((_K,)), pltpu.SemaphoreType.DMA],
        out_shape=[jax.ShapeDtypeStruct((_N,), jnp.float32),
                   jax.ShapeDtypeStruct((512 * 64,), jnp.float32)],
    )(e1, r1)
    return (eo.reshape(100000, 64), ro.reshape(512, 64))


# speedup vs baseline: 7.4871x; 7.4871x over previous
import functools

import jax
import jax.numpy as jnp
from jax import lax
from jax.experimental import pallas as pl
from jax.experimental.pallas import tpu as pltpu
from jax.experimental.pallas import tpu_sc as plsc

_NUM_ENTITIES = 100000
_NUM_RELATIONS = 512
_EMB_DIM = 64
_NC = 2
_NS = 16
_NW = _NC * _NS
_SC_ROWS = 76800
_CHUNK = 400
_NFULL = _SC_ROWS // _CHUNK
_ROUNDS = _NFULL // _NW
_REL_PER_W = _NUM_RELATIONS // _NW
_TC_ROWS = _NUM_ENTITIES - _SC_ROWS
_TC_BLOCK = 800


def _sc_copy_body(ent_in, rel_in, ent_out, rel_out,
                  buf0, buf1, rbuf, sin0, sin1, sout0, sout1):
    wid = lax.axis_index("s") * _NC + lax.axis_index("c")
    bufs = (buf0, buf1)
    in_sems = (sin0, sin1)
    out_sems = (sout0, sout1)

    out_copies = [None] * _ROUNDS
    for k in range(_ROUNDS):
        b = k % 2
        if k >= 2:
            out_copies[k - 2].wait()
        rows = pl.ds((wid + k * _NW) * _CHUNK, _CHUNK)
        in_copy = pltpu.make_async_copy(ent_in.at[rows], bufs[b], in_sems[b])
        in_copy.start()
        in_copy.wait()
        out_copies[k] = pltpu.make_async_copy(bufs[b], ent_out.at[rows],
                                              out_sems[b])
        out_copies[k].start()

    rrows = pl.ds(wid * _REL_PER_W, _REL_PER_W)
    pltpu.sync_copy(rel_in.at[rrows], rbuf)
    pltpu.sync_copy(rbuf, rel_out.at[rrows])

    out_copies[_ROUNDS - 2].wait()
    out_copies[_ROUNDS - 1].wait()


def _sc_copy(entity_emb, rel_emb):
    mesh = plsc.VectorSubcoreMesh(core_axis_name="c", subcore_axis_name="s")
    k = pl.kernel(
        _sc_copy_body,
        out_type=[
            jax.ShapeDtypeStruct((_NUM_ENTITIES, _EMB_DIM), jnp.float32),
            jax.ShapeDtypeStruct((_NUM_RELATIONS, _EMB_DIM), jnp.float32),
        ],
        mesh=mesh,
        scratch_types=[
            pltpu.VMEM((_CHUNK, _EMB_DIM), jnp.float32),
            pltpu.VMEM((_CHUNK, _EMB_DIM), jnp.float32),
            pltpu.VMEM((_REL_PER_W, _EMB_DIM), jnp.float32),
            pltpu.SemaphoreType.DMA,
            pltpu.SemaphoreType.DMA,
            pltpu.SemaphoreType.DMA,
            pltpu.SemaphoreType.DMA,
        ],
    )
    return k(entity_emb, rel_emb)


def _tc_body(x_ref, o_ref):
    o_ref[...] = x_ref[...]


def _tc_tail_copy(entity_emb):
    return pl.pallas_call(
        _tc_body,
        grid=(_TC_ROWS // _TC_BLOCK,),
        in_specs=[pl.BlockSpec((_TC_BLOCK, _EMB_DIM),
                               lambda i: (i + _SC_ROWS // _TC_BLOCK, 0))],
        out_specs=pl.BlockSpec((_TC_BLOCK, _EMB_DIM), lambda i: (i, 0)),
        out_shape=jax.ShapeDtypeStruct((_TC_ROWS, _EMB_DIM), jnp.float32),
    )(entity_emb)


def kernel(x_dict, edge_index, entity_emb, rel_emb):
    ent_sc, rel_out = _sc_copy(entity_emb, rel_emb)
    tc_part = _tc_tail_copy(entity_emb)
    entity_out = lax.dynamic_update_slice(ent_sc, tc_part, (_SC_ROWS, 0))
    return (entity_out, rel_out)


# speedup vs baseline: 8.4773x; 1.1322x over previous
import jax
import jax.numpy as jnp
from jax.experimental import pallas as pl
from jax.experimental.pallas import tpu as pltpu

_CHUNK = 1000
_K = 10
_R = 10


def _copy_body(ent_in, rel_in, ent_out, rel_out, bufs, rbuf,
               in_sems, out_sems, rsem):
    rin = pltpu.make_async_copy(rel_in, rbuf, rsem)
    rin.start()

    def rows(c, r):
        return pl.ds((r * _K + c) * _CHUNK, _CHUNK)

    in_copies, out_copies = {}, {}
    for c in range(_K):
        in_copies[(c, 0)] = pltpu.make_async_copy(
            ent_in.at[rows(c, 0)], bufs.at[c, 0], in_sems.at[c, 0])
        in_copies[(c, 0)].start(priority=c % 2)
    for r in range(_R):
        b, nb = r % 2, (r + 1) % 2
        for c in range(_K):
            if r + 1 < _R:
                if r >= 1:
                    out_copies[(c, r - 1)].wait()
                in_copies[(c, r + 1)] = pltpu.make_async_copy(
                    ent_in.at[rows(c, r + 1)], bufs.at[c, nb],
                    in_sems.at[c, nb])
                in_copies[(c, r + 1)].start(priority=c % 2)
            in_copies[(c, r)].wait()
            out_copies[(c, r)] = pltpu.make_async_copy(
                bufs.at[c, b], ent_out.at[rows(c, r)], out_sems.at[c, b])
            out_copies[(c, r)].start(priority=c % 2)

    rin.wait()
    rout = pltpu.make_async_copy(rbuf, rel_out, rsem)
    rout.start()
    for c in range(_K):
        out_copies[(c, _R - 2)].wait()
        out_copies[(c, _R - 1)].wait()
    rout.wait()


def kernel(x_dict, edge_index, entity_emb, rel_emb):
    entity_out, rel_out = pl.pallas_call(
        _copy_body,
        in_specs=[
            pl.BlockSpec(memory_space=pl.ANY),
            pl.BlockSpec(memory_space=pl.ANY),
        ],
        out_specs=[
            pl.BlockSpec(memory_space=pl.ANY),
            pl.BlockSpec(memory_space=pl.ANY),
        ],
        scratch_shapes=[
            pltpu.VMEM((_K, 2, _CHUNK, 64), jnp.float32),
            pltpu.VMEM((512, 64), jnp.float32),
            pltpu.SemaphoreType.DMA((_K, 2)),
            pltpu.SemaphoreType.DMA((_K, 2)),
            pltpu.SemaphoreType.DMA,
        ],
        out_shape=[
            jax.ShapeDtypeStruct(entity_emb.shape, entity_emb.dtype),
            jax.ShapeDtypeStruct(rel_emb.shape, rel_emb.dtype),
        ],
    )(entity_emb, rel_emb)
    return (entity_out, rel_out)
